# Initial kernel scaffold; baseline (speedup 1.0000x reference)
import jax
import jax.numpy as jnp
from jax.experimental import pallas as pl


def kernel(input_batch, emb_table, W1, b1, W2, b2):
    raise NotImplementedError("write your pallas kernel here")


# speedup vs baseline: 6.7418x; 6.7418x over previous
import functools

import jax
import jax.numpy as jnp
from jax import lax
from jax.experimental import pallas as pl
from jax.experimental.pallas import tpu as pltpu
from jax.experimental.pallas import tpu_sc as plsc

VOCAB = 100000
EMBED = 128
HIDDEN = 512
OUT = 256
BATCH = 4096
HIST = 50

NC = 2
NS = 16
NW = NC * NS
ROWS_PER_W = BATCH // NW
CB = 2
CHUNK_IDX = CB * HIST
NCHUNK = ROWS_PER_W // CB
LANES = 16
EV = EMBED // LANES

_sc_mesh = plsc.VectorSubcoreMesh(
    core_axis_name="c", subcore_axis_name="s", num_cores=NC, num_subcores=NS
)


@functools.partial(
    pl.kernel,
    out_type=jax.ShapeDtypeStruct((BATCH, EMBED), jnp.float32),
    mesh=_sc_mesh,
    scratch_types=[
        pltpu.VMEM((NCHUNK, CHUNK_IDX), jnp.int32),
        pltpu.VMEM((CHUNK_IDX, EMBED), jnp.float32),
        pltpu.VMEM((CB, EMBED), jnp.float32),
        pltpu.SemaphoreType.DMA,
    ],
)
def _embbag_sum(idx_hbm, table_hbm, out_hbm, idx_v, rows_v, pout_v, sem):
    wid = lax.axis_index("s") * NC + lax.axis_index("c")

    pltpu.sync_copy(idx_hbm.at[pl.ds(wid * NCHUNK, NCHUNK)], idx_v)

    def chunk_body(c, carry):
        pltpu.async_copy(table_hbm.at[idx_v.at[c]], rows_v, sem).wait()
        for i in range(CB):
            def bag_body(r, accs):
                return tuple(
                    accs[j] + rows_v[i * HIST + r, pl.ds(j * LANES, LANES)]
                    for j in range(EV)
                )
            accs = lax.fori_loop(
                0, HIST, bag_body,
                tuple(jnp.zeros((LANES,), jnp.float32) for _ in range(EV)),
            )
            for j in range(EV):
                pout_v[i, pl.ds(j * LANES, LANES)] = accs[j]
        pltpu.sync_copy(
            pout_v, out_hbm.at[pl.ds(wid * ROWS_PER_W + c * CB, CB)]
        )
        return carry

    lax.fori_loop(0, NCHUNK, chunk_body, 0)


def _mlp_body(x_ref, w1_ref, b1_ref, w2_ref, b2_ref, o_ref):
    x = x_ref[...] * (1.0 / HIST)
    h = lax.dot_general(
        x, w1_ref[...], (((1,), (1,)), ((), ())),
        preferred_element_type=jnp.float32,
    ) + b1_ref[...]
    o_ref[...] = lax.dot_general(
        h, w2_ref[...], (((1,), (1,)), ((), ())),
        preferred_element_type=jnp.float32,
    ) + b2_ref[...]


_mlp = pl.pallas_call(
    _mlp_body,
    out_shape=jax.ShapeDtypeStruct((BATCH, OUT), jnp.float32),
)


@jax.jit
def kernel(input_batch, emb_table, W1, b1, W2, b2):
    idx2d = input_batch.astype(jnp.int32).reshape(BATCH // CB, CHUNK_IDX)
    pooled_sum = _embbag_sum(idx2d, emb_table)
    return _mlp(pooled_sum, W1, b1.reshape(1, HIDDEN), W2, b2.reshape(1, OUT))
